# baseline (speedup 1.0000x reference)
import jax
import jax.numpy as jnp
from jax.experimental import pallas as pl
from jax.experimental.pallas import tpu as pltpu

_MB = 1024 * 1024


def _nsplit_kernel(x_ref, w0_ref, w1_ref, b_ref, o_ref):
    j = pl.program_id(1)
    fo = w0_ref.shape[1]

    @pl.when(j == 0)
    def _():
        acc = jnp.dot(x_ref[...], w0_ref[...],
                      preferred_element_type=jnp.float32)
        o_ref[...] = (acc + b_ref[:, :fo].astype(jnp.float32)).astype(o_ref.dtype)

    @pl.when(j == 1)
    def _():
        acc = jnp.dot(x_ref[...], w1_ref[...],
                      preferred_element_type=jnp.float32)
        o_ref[...] = (acc + b_ref[:, fo:].astype(jnp.float32)).astype(o_ref.dtype)


def _plain_kernel(x_ref, w_ref, b_ref, o_ref):
    acc = jnp.dot(x_ref[...], w_ref[...], preferred_element_type=jnp.float32)
    o_ref[...] = (acc + b_ref[...].astype(jnp.float32)).astype(o_ref.dtype)


def _pick_tm(n):
    for tm in (2048, 1024, 512, 256, 128, 64, 32, 16, 8):
        if n % tm == 0:
            return tm
    return None


def _linear2d(x2, w, b2):
    n, f_in = x2.shape
    f_out = w.shape[1]
    dtype = x2.dtype
    itemsize = jnp.dtype(dtype).itemsize

    tm = _pick_tm(n)
    if tm is None:
        tm = min(n, 1024)

    cost = pl.CostEstimate(
        flops=2 * n * f_in * f_out,
        transcendentals=0,
        bytes_accessed=itemsize * (n * f_in + f_in * f_out + f_out + n * f_out),
    )

    if f_out % 256 == 0 and n % tm == 0:
        fo = f_out // 2
        grid = (n // tm, 2)
        return pl.pallas_call(
            _nsplit_kernel,
            out_shape=jax.ShapeDtypeStruct((n, f_out), dtype),
            grid=grid,
            in_specs=[
                pl.BlockSpec((tm, f_in), lambda i, j: (i, 0)),
                pl.BlockSpec((f_in, fo), lambda i, j: (0, 0)),
                pl.BlockSpec((f_in, fo), lambda i, j: (0, 1)),
                pl.BlockSpec((1, f_out), lambda i, j: (0, 0)),
            ],
            out_specs=pl.BlockSpec((tm, fo), lambda i, j: (i, j)),
            compiler_params=pltpu.CompilerParams(
                dimension_semantics=("parallel", "arbitrary"),
                vmem_limit_bytes=56 * _MB,
            ),
            cost_estimate=cost,
        )(x2, w, w, b2)

    grid = (pl.cdiv(n, tm),)
    return pl.pallas_call(
        _plain_kernel,
        out_shape=jax.ShapeDtypeStruct((n, f_out), dtype),
        grid=grid,
        in_specs=[
            pl.BlockSpec((tm, f_in), lambda i: (i, 0)),
            pl.BlockSpec((f_in, f_out), lambda i: (0, 0)),
            pl.BlockSpec((1, f_out), lambda i: (0, 0)),
        ],
        out_specs=pl.BlockSpec((tm, f_out), lambda i: (i, 0)),
        compiler_params=pltpu.CompilerParams(
            dimension_semantics=("parallel",),
            vmem_limit_bytes=56 * _MB,
        ),
        cost_estimate=cost,
    )(x2, w, b2)


def kernel(x, w, b):
    f_out = w.shape[1]
    b2 = b.reshape(1, f_out)
    if x.ndim <= 2:
        x2 = x.reshape(1, -1) if x.ndim == 1 else x
        y = _linear2d(x2, w, b2)
        return y.reshape(-1) if x.ndim == 1 else y
    x2 = x.reshape(-1, x.shape[-1])
    y = _linear2d(x2, w, b2)
    return y.reshape(-1, x.shape[1], f_out)


# speedup vs baseline: 1.4455x; 1.4455x over previous
import functools

import jax
import jax.numpy as jnp
from jax.experimental import pallas as pl
from jax.experimental.pallas import tpu as pltpu

_MB = 1024 * 1024
_CM = 1024
_XBUF = 3
_OBUF = 4


def _manual_kernel(nsteps, x_hbm, w_hbm, b_hbm, o_hbm,
                   x_buf, w_buf, b_buf, o_buf,
                   sem_x, sem_w, sem_b, sem_o):
    i = pl.program_id(0)

    def x_copy(chunk, slot):
        return pltpu.make_async_copy(
            x_hbm.at[pl.ds(chunk * _CM, _CM), :], x_buf.at[slot],
            sem_x.at[slot])

    def o_copy(chunk, slot):
        return pltpu.make_async_copy(
            o_buf.at[slot], o_hbm.at[pl.ds(chunk * _CM, _CM), :],
            sem_o.at[slot])

    _H = _CM // 2

    def o_copy_half(chunk, slot, half):
        return pltpu.make_async_copy(
            o_buf.at[slot, pl.ds(half * _H, _H), :],
            o_hbm.at[pl.ds(chunk * _CM + half * _H, _H), :],
            sem_o.at[slot])

    @pl.when(i == 0)
    def _():
        pltpu.make_async_copy(w_hbm, w_buf, sem_w).start()
        pltpu.make_async_copy(b_hbm, b_buf, sem_b).start()
        for c in range(min(_XBUF, nsteps)):
            x_copy(c, c).start()

    @pl.when(i == 0)
    def _():
        pltpu.make_async_copy(w_hbm, w_buf, sem_w).wait()
        pltpu.make_async_copy(b_hbm, b_buf, sem_b).wait()

    x_copy(i, i % _XBUF).wait()
    @pl.when(i >= _OBUF)
    def _():
        o_copy(i - _OBUF, i % _OBUF).wait()

    last = nsteps - 1
    slot = i % _OBUF
    xs = x_buf.at[i % _XBUF]
    acc0 = jnp.dot(xs[:_H], w_buf[...], preferred_element_type=jnp.float32)
    o_buf[slot, :_H, :] = acc0 + b_buf[0:1, :]

    @pl.when(i == last)
    def _():
        o_copy_half(i, slot, 0).start()

    acc1 = jnp.dot(xs[_H:], w_buf[...], preferred_element_type=jnp.float32)
    o_buf[slot, _H:, :] = acc1 + b_buf[0:1, :]

    @pl.when(i == last)
    def _():
        o_copy_half(i, slot, 1).start()

    @pl.when(i != last)
    def _():
        o_copy(i, slot).start()

    @pl.when(i + _XBUF < nsteps)
    def _():
        x_copy(i + _XBUF, i % _XBUF).start()

    @pl.when(i == nsteps - 1)
    def _():
        for k in range(min(_OBUF - 1, nsteps - 1)):
            o_copy(i - 1 - k, (i - 1 - k) % _OBUF).wait()
        o_copy_half(i, i % _OBUF, 0).wait()
        o_copy_half(i, i % _OBUF, 1).wait()


def _plain_kernel(x_ref, w_ref, b_ref, o_ref):
    acc = jnp.dot(x_ref[...], w_ref[...], preferred_element_type=jnp.float32)
    o_ref[...] = (acc + b_ref[...].astype(jnp.float32)).astype(o_ref.dtype)


def _linear2d(x2, w, b2):
    n, f_in = x2.shape
    f_out = w.shape[1]
    dtype = x2.dtype
    itemsize = jnp.dtype(dtype).itemsize

    cost = pl.CostEstimate(
        flops=2 * n * f_in * f_out,
        transcendentals=0,
        bytes_accessed=itemsize * (n * f_in + f_in * f_out + f_out + n * f_out),
    )

    if dtype == jnp.float32 and n % _CM == 0 and n // _CM >= 2:
        nsteps = n // _CM
        return pl.pallas_call(
            functools.partial(_manual_kernel, nsteps),
            out_shape=jax.ShapeDtypeStruct((n, f_out), dtype),
            grid=(nsteps,),
            in_specs=[
                pl.BlockSpec(memory_space=pl.ANY),
                pl.BlockSpec(memory_space=pl.ANY),
                pl.BlockSpec(memory_space=pl.ANY),
            ],
            out_specs=pl.BlockSpec(memory_space=pl.ANY),
            scratch_shapes=[
                pltpu.VMEM((_XBUF, _CM, f_in), dtype),
                pltpu.VMEM((f_in, f_out), dtype),
                pltpu.VMEM((1, f_out), dtype),
                pltpu.VMEM((_OBUF, _CM, f_out), jnp.float32),
                pltpu.SemaphoreType.DMA((_XBUF,)),
                pltpu.SemaphoreType.DMA,
                pltpu.SemaphoreType.DMA,
                pltpu.SemaphoreType.DMA((_OBUF,)),
            ],
            compiler_params=pltpu.CompilerParams(
                dimension_semantics=("arbitrary",),
                vmem_limit_bytes=56 * _MB,
            ),
            cost_estimate=cost,
        )(x2, w, b2)

    tm = _CM
    for cand in (2048, 1024, 512, 256, 128, 64, 32, 16, 8):
        if n % cand == 0:
            tm = cand
            break
    else:
        tm = min(n, 1024)
    grid = (pl.cdiv(n, tm),)
    return pl.pallas_call(
        _plain_kernel,
        out_shape=jax.ShapeDtypeStruct((n, f_out), dtype),
        grid=grid,
        in_specs=[
            pl.BlockSpec((tm, f_in), lambda i: (i, 0)),
            pl.BlockSpec((f_in, f_out), lambda i: (0, 0)),
            pl.BlockSpec((1, f_out), lambda i: (0, 0)),
        ],
        out_specs=pl.BlockSpec((tm, f_out), lambda i: (i, 0)),
        compiler_params=pltpu.CompilerParams(
            dimension_semantics=("parallel",),
            vmem_limit_bytes=56 * _MB,
        ),
        cost_estimate=cost,
    )(x2, w, b2)


def kernel(x, w, b):
    f_out = w.shape[1]
    b2 = b.reshape(1, f_out)
    if x.ndim <= 2:
        x2 = x.reshape(1, -1) if x.ndim == 1 else x
        y = _linear2d(x2, w, b2)
        return y.reshape(-1) if x.ndim == 1 else y
    x2 = x.reshape(-1, x.shape[-1])
    y = _linear2d(x2, w, b2)
    return y.reshape(-1, x.shape[1], f_out)
